# baseline (speedup 1.0000x reference)
import jax
import jax.numpy as jnp
from jax import lax
from jax.experimental import pallas as pl
from jax.experimental.pallas import tpu as pltpu
from jax.experimental.pallas import tpu_sc as plsc

_B, _H, _L1 = 1024, 200, 201
_V, _D, _R = 1000000, 64, 6
_N = _B * _H

_NC, _NS = 2, 16
_NW = _NC * _NS
_RPW = _N // _NW
_CH = 128
_NCH = _RPW // _CH
_NB = 2


def _sc_gather_body(ct_hbm, et_hbm, cidx_hbm, eidx_hbm, gc_hbm, ge_hbm,
                    cidx_v, eidx_v, buf0, buf1, gsem, wsem0, wsem1):
    wid = lax.axis_index("s") * _NC + lax.axis_index("c")
    base = wid * _RPW
    pltpu.sync_copy(cidx_hbm.at[pl.ds(base, _RPW)], cidx_v)
    pltpu.sync_copy(eidx_hbm.at[pl.ds(base, _RPW)], eidx_v)
    bufs = (buf0, buf1)
    wsems = (wsem0, wsem1)

    def run_branch(tab_hbm, idx_v, out_hbm, drain_shape_ref):
        def superchunk(s, carry):
            for b in range(_NB):
                c = s * _NB + b

                @pl.when(s > 0)
                def _():
                    pltpu.make_async_copy(
                        bufs[b], out_hbm.at[pl.ds(base, _CH)], wsems[b]).wait()

                idx_view = idx_v.at[pl.ds(c * _CH, _CH)]
                pltpu.async_copy(tab_hbm.at[idx_view], bufs[b], gsem).wait()
                pltpu.async_copy(
                    bufs[b], out_hbm.at[pl.ds(base + c * _CH, _CH)], wsems[b])
            return carry

        lax.fori_loop(0, _NCH // _NB, superchunk, 0)
        for b in range(_NB):
            pltpu.make_async_copy(
                bufs[b], drain_shape_ref.at[pl.ds(base, _CH)], wsems[b]).wait()

    run_branch(ct_hbm, cidx_v, gc_hbm, gc_hbm)
    run_branch(et_hbm, eidx_v, ge_hbm, ge_hbm)


import functools


@functools.cache
def _make_sc_gather():
    return pl.kernel(
        _sc_gather_body,
        out_type=(
            jax.ShapeDtypeStruct((_N, _D), jnp.float32),
            jax.ShapeDtypeStruct((_N, _D), jnp.float32),
        ),
        mesh=plsc.VectorSubcoreMesh(core_axis_name="c", subcore_axis_name="s",
                                    num_cores=_NC, num_subcores=_NS),
        compiler_params=pltpu.CompilerParams(use_tc_tiling_on_sc=False),
        scratch_types=[
            pltpu.VMEM((_RPW,), jnp.int32),
            pltpu.VMEM((_RPW,), jnp.int32),
            pltpu.VMEM((_CH, _D), jnp.float32),
            pltpu.VMEM((_CH, _D), jnp.float32),
            pltpu.SemaphoreType.DMA,
            pltpu.SemaphoreType.DMA,
            pltpu.SemaphoreType.DMA,
        ],
    )


def _sc_gather(ct, et, cidx, eidx):
    return _make_sc_gather()(ct, et, cidx, eidx)

_RT = 2048


def _tc_post_body(gc_ref, ge_ref, cgap_ref, egap_ref, erat_ref,
                  wc_ref, we_ref, cu_ref, cc_ref, eu_ref, ce_ref, rp_ref,
                  co_ref, eo_ref):
    lgc = jnp.log(cgap_ref[...] + 1.0)
    co_ref[...] = (
        jnp.dot(gc_ref[...], wc_ref[...], preferred_element_type=jnp.float32)
        + lgc * cu_ref[...] + cc_ref[...])
    lge = jnp.log(egap_ref[...] + 1.0)
    onehot = (erat_ref[...] == lax.broadcasted_iota(
        jnp.int32, (_RT, 8), 1)).astype(jnp.float32)
    eo_ref[...] = (
        jnp.dot(ge_ref[...], we_ref[...], preferred_element_type=jnp.float32)
        + lge * eu_ref[...]
        + jnp.dot(onehot, rp_ref[...], preferred_element_type=jnp.float32)
        + ce_ref[...])


def _tc_post(gc, ge, cgap, egap, erat, wc, we, cu, cc, eu, ce, rp):
    n_blk = _N // _RT
    row_spec = pl.BlockSpec((_RT, _D), lambda i: (i, 0))
    col_spec = pl.BlockSpec((_RT, 1), lambda i: (i, 0))

    def small(shape):
        return pl.BlockSpec(shape, lambda i: (0, 0))

    return pl.pallas_call(
        _tc_post_body,
        grid=(n_blk,),
        in_specs=[
            row_spec, row_spec, col_spec, col_spec, col_spec,
            small((_D, _D)), small((_D, _D)), small((1, _D)), small((1, _D)),
            small((1, _D)), small((1, _D)), small((8, _D)),
        ],
        out_specs=[row_spec, row_spec],
        out_shape=[
            jax.ShapeDtypeStruct((_N, _D), jnp.float32),
            jax.ShapeDtypeStruct((_N, _D), jnp.float32),
        ],
    )(gc, ge, cgap, egap, erat, wc, we, cu, cc, eu, ce, rp)


def kernel(row0, row1, row2, row3, row4, row5, row6, row7, click_table,
           exposure_table, rating_table, ts_w, ts_b, exp_w, exp_b, clk_w,
           clk_b):
    del row2, row3, row7
    item_time = row6[:, -1]
    cgap = (item_time[:, None] - row1).reshape(_N, 1)
    egap = (item_time[:, None] - row6[:, :-1]).reshape(_N, 1)
    erat = row5[:, :-1].reshape(_N, 1).astype(jnp.int32)
    cidx = row0.reshape(_N).astype(jnp.int32)
    eidx = row4[:, :-1].reshape(_N).astype(jnp.int32)

    wc = clk_w[:_D]
    we = exp_w[:_D]
    cu = ts_w @ clk_w[_D:2 * _D]
    cc = (ts_b @ clk_w[_D:2 * _D] + rating_table[2] @ clk_w[2 * _D:]
          + clk_b)[None, :]
    eu = ts_w @ exp_w[_D:2 * _D]
    ce = (ts_b @ exp_w[_D:2 * _D] + exp_b)[None, :]
    rp = jnp.zeros((8, _D), jnp.float32).at[:_R].set(
        rating_table @ exp_w[2 * _D:])

    gc, ge = _sc_gather(click_table, exposure_table, cidx, eidx)
    co, eo = _tc_post(gc, ge, cgap, egap, erat, wc, we, cu, cc, eu, ce, rp)
    return co.reshape(_B, _H, _D), eo.reshape(_B, _H, _D)
